# Initial kernel scaffold; baseline (speedup 1.0000x reference)
import jax
import jax.numpy as jnp
from jax.experimental import pallas as pl


def kernel(x, edge_index, W0, b0, W_rest, b_rest, Wr1, br1, Wr2, br2):
    raise NotImplementedError("write your pallas kernel here")


# speedup vs baseline: 5.1408x; 5.1408x over previous
import functools

import jax
import jax.numpy as jnp
from jax import lax
from jax.experimental import pallas as pl
from jax.experimental.pallas import tpu as pltpu
from jax.experimental.pallas import tpu_sc as plsc

_N = 10000
_E = 160000
_DIN = 256
_H = 512
_OUT = 128
_L = 10

_NC, _NS = 2, 16
_BLK = 128
_NBLK_T = 80
_NBH = _NBLK_T // 2
_EPAD = _NS * _NBLK_T * _BLK
_ACC_ROWS = 10112
_ROWS = 1000

@functools.cache
def _mesh():
    return plsc.VectorSubcoreMesh(core_axis_name="c", subcore_axis_name="s",
                                  num_cores=_NC, num_subcores=_NS)


def _sc_degree(pk):
    @functools.partial(
        pl.kernel,
        out_type=jax.ShapeDtypeStruct((_NC * _N, 128), jnp.float32),
        mesh=_mesh(),
        scratch_types=[
            pltpu.VMEM((_NBH, _BLK), jnp.int32),
            pltpu.VMEM((_BLK, 128), jnp.float32),
            pltpu.VMEM_SHARED((_ACC_ROWS, 128), jnp.float32),
        ],
    )
    def k(pk_hbm, deg_hbm, colv, fb, accd):
        c = lax.axis_index("c")
        s = lax.axis_index("s")
        pltpu.sync_copy(pk_hbm.at[pl.ds(s * _NBLK_T + c * _NBH, _NBH)], colv)
        ov = jnp.ones((16,), jnp.float32)
        zv = jnp.zeros((16,), jnp.float32)

        @pl.loop(0, _NBH)
        def _unpack(i):
            for j in range(_BLK // 16):
                sl = pl.ds(j * 16, 16)
                colv[i, sl] = lax.shift_right_logical(colv[i, sl], 14)

        @pl.loop(0, _BLK)
        def _fill0(i):
            for j in range(8):
                fb[i, pl.ds(j * 16, 16)] = zv

        for j in range(4):
            pltpu.sync_copy(fb, accd.at[pl.ds(s * 632 + j * 128, 128)])
        pltpu.sync_copy(fb.at[pl.ds(0, 120)],
                        accd.at[pl.ds(s * 632 + 512, 120)])

        @pl.loop(0, _BLK)
        def _fill1(i):
            for j in range(8):
                fb[i, pl.ds(j * 16, 16)] = ov

        plsc.subcore_barrier()

        @pl.loop(0, _NBH)
        def _scat(b):
            pltpu.sync_copy(fb, accd.at[colv.at[b]], add=True)

        plsc.subcore_barrier()
        pltpu.sync_copy(accd.at[pl.ds(s * 624, 624)],
                        deg_hbm.at[pl.ds(c * _N + s * 624, 624)])

        @pl.when(s == _NS - 1)
        def _tail():
            pltpu.sync_copy(accd.at[pl.ds(_N - 16, 16)],
                            deg_hbm.at[pl.ds(c * _N + _N - 16, 16)])

    return k(pk)


def _sc_scatter(u_flat, pk):
    @functools.partial(
        pl.kernel,
        out_type=jax.ShapeDtypeStruct((4 * _N, 128), jnp.float32),
        mesh=_mesh(),
        scratch_types=[
            pltpu.VMEM((_NBH, _BLK), jnp.int32),
            pltpu.VMEM((_NBH, _BLK), jnp.int32),
            pltpu.VMEM((_BLK, 128), jnp.float32),
            pltpu.VMEM((_BLK, 128), jnp.float32),
            pltpu.VMEM_SHARED((_ACC_ROWS, 128), jnp.float32),
            pltpu.SemaphoreType.DMA,
            pltpu.SemaphoreType.DMA,
        ],
    )
    def k(u_hbm, pk_hbm, s_hbm,
          colv, rowv, gb0, gb1, acc, sem0, sem1):
        c = lax.axis_index("c")
        s = lax.axis_index("s")
        zv = jnp.zeros((16,), jnp.float32)
        mask = jnp.full((16,), 16383, jnp.int32)

        for p in range(2):
            chunk = 2 * c + p
            roff = chunk * _N

            @pl.loop(0, _BLK)
            def _fill0(i):
                for j in range(8):
                    gb0[i, pl.ds(j * 16, 16)] = zv

            for j in range(4):
                pltpu.sync_copy(gb0, acc.at[pl.ds(s * 632 + j * 128, 128)])
            pltpu.sync_copy(gb0.at[pl.ds(0, 120)],
                            acc.at[pl.ds(s * 632 + 512, 120)])
            plsc.subcore_barrier()

            for half in range(2):
                pltpu.sync_copy(
                    pk_hbm.at[pl.ds(s * _NBLK_T + half * _NBH, _NBH)], colv)

                @pl.loop(0, _NBH)
                def _unpack(i):
                    for j in range(_BLK // 16):
                        sl = pl.ds(j * 16, 16)
                        v = colv[i, sl]
                        rowv[i, sl] = lax.bitwise_and(v, mask) + roff
                        colv[i, sl] = lax.shift_right_logical(v, 14)

                pltpu.async_copy(u_hbm.at[rowv.at[0]], gb0, sem0)
                pltpu.async_copy(u_hbm.at[rowv.at[1]], gb1, sem1)

                @pl.loop(0, _NBH // 2)
                def _body(it):
                    b0 = it * 2
                    b1 = b0 + 1
                    pltpu.make_async_copy(u_hbm.at[rowv.at[b0]], gb0,
                                          sem0).wait()
                    pltpu.sync_copy(gb0, acc.at[colv.at[b0]], add=True)
                    pltpu.async_copy(
                        u_hbm.at[rowv.at[lax.rem(b0 + 2, _NBH)]], gb0, sem0)
                    pltpu.make_async_copy(u_hbm.at[rowv.at[b1]], gb1,
                                          sem1).wait()
                    pltpu.sync_copy(gb1, acc.at[colv.at[b1]], add=True)
                    pltpu.async_copy(
                        u_hbm.at[rowv.at[lax.rem(b1 + 2, _NBH)]], gb1, sem1)

                pltpu.make_async_copy(u_hbm.at[rowv.at[0]], gb0, sem0).wait()
                pltpu.make_async_copy(u_hbm.at[rowv.at[1]], gb1, sem1).wait()

            plsc.subcore_barrier()
            pltpu.sync_copy(acc.at[pl.ds(s * 624, 624)],
                            s_hbm.at[pl.ds(chunk * _N + s * 624, 624)])

            @pl.when(s == _NS - 1)
            def _tail():
                pltpu.sync_copy(acc.at[pl.ds(_N - 16, 16)],
                                s_hbm.at[pl.ds(chunk * _N + _N - 16, 16)])

            plsc.subcore_barrier()

    return k(u_flat, pk)


def _prep_body(x_ref, w_ref, degs_ref, u_ref, dis_ref):
    deg = degs_ref[0, :, 0] + degs_ref[1, :, 0] + 1.0
    dis = lax.rsqrt(deg)
    t = jnp.dot(x_ref[...], w_ref[...], preferred_element_type=jnp.float32)
    u = t * dis[:, None]
    for cc in range(4):
        u_ref[cc] = u[:, cc * 128:(cc + 1) * 128]
    dis_ref[...] = dis[:, None]


def _tc_prep(x, w0a, deg):
    return pl.pallas_call(
        _prep_body,
        grid=(_N // _ROWS,),
        in_specs=[
            pl.BlockSpec((_ROWS, _DIN), lambda i: (i, 0)),
            pl.BlockSpec((_DIN, _H), lambda i: (0, 0)),
            pl.BlockSpec((2, _ROWS, 128), lambda i: (0, i, 0)),
        ],
        out_specs=[
            pl.BlockSpec((4, _ROWS, 128), lambda i: (0, i, 0)),
            pl.BlockSpec((_ROWS, 1), lambda i: (i, 0)),
        ],
        out_shape=[
            jax.ShapeDtypeStruct((4, _N, 128), jnp.float32),
            jax.ShapeDtypeStruct((_N, 1), jnp.float32),
        ],
    )(x, w0a, deg)


def _layer_body(s_ref, u_ref, dis_ref, b_ref, w_ref, o_ref):
    dis = dis_ref[...]
    acc = jnp.zeros((_ROWS, _H), jnp.float32)
    for cc in range(4):
        h_c = jnp.maximum(
            (s_ref[cc] + u_ref[cc]) * dis + b_ref[0, cc * 128:(cc + 1) * 128],
            0.0)
        acc = acc + jnp.dot(h_c, w_ref[cc],
                            preferred_element_type=jnp.float32)
    t = acc * dis
    for cc in range(4):
        o_ref[cc] = t[:, cc * 128:(cc + 1) * 128]


def _tc_layer(s, u, dis, b, w):
    return pl.pallas_call(
        _layer_body,
        grid=(_N // _ROWS,),
        in_specs=[
            pl.BlockSpec((4, _ROWS, 128), lambda i: (0, i, 0)),
            pl.BlockSpec((4, _ROWS, 128), lambda i: (0, i, 0)),
            pl.BlockSpec((_ROWS, 1), lambda i: (i, 0)),
            pl.BlockSpec((1, _H), lambda i: (0, 0)),
            pl.BlockSpec((4, 128, _H), lambda i: (0, 0, 0)),
        ],
        out_specs=pl.BlockSpec((4, _ROWS, 128), lambda i: (0, i, 0)),
        out_shape=jax.ShapeDtypeStruct((4, _N, 128), jnp.float32),
    )(s, u, dis, b.reshape(1, _H), w)


def _final_body(s_ref, u_ref, dis_ref, b_ref, w1_ref, b1_ref, w2_ref, b2_ref,
                o_ref):
    dis = dis_ref[...]
    acc = jnp.zeros((_ROWS, _H), jnp.float32)
    for cc in range(4):
        h_c = jnp.maximum(
            (s_ref[cc] + u_ref[cc]) * dis + b_ref[0, cc * 128:(cc + 1) * 128],
            0.0)
        acc = acc + jnp.dot(h_c, w1_ref[cc],
                            preferred_element_type=jnp.float32)
    g = jnp.maximum(acc + b1_ref[...], 0.0)
    o_ref[...] = jnp.dot(g, w2_ref[...],
                         preferred_element_type=jnp.float32) + b2_ref[...]


def _tc_final(s, u, dis, b, w1, b1, w2, b2):
    return pl.pallas_call(
        _final_body,
        grid=(_N // _ROWS,),
        in_specs=[
            pl.BlockSpec((4, _ROWS, 128), lambda i: (0, i, 0)),
            pl.BlockSpec((4, _ROWS, 128), lambda i: (0, i, 0)),
            pl.BlockSpec((_ROWS, 1), lambda i: (i, 0)),
            pl.BlockSpec((1, _H), lambda i: (0, 0)),
            pl.BlockSpec((4, 128, _H), lambda i: (0, 0, 0)),
            pl.BlockSpec((1, _H), lambda i: (0, 0)),
            pl.BlockSpec((_H, _OUT), lambda i: (0, 0)),
            pl.BlockSpec((1, _OUT), lambda i: (0, 0)),
        ],
        out_specs=pl.BlockSpec((_ROWS, _OUT), lambda i: (i, 0)),
        out_shape=jax.ShapeDtypeStruct((_N, _OUT), jnp.float32),
    )(s, u, dis, b.reshape(1, _H), w1, b1.reshape(1, _H), w2,
      b2.reshape(1, _OUT))


def kernel(x, edge_index, W0, b0, W_rest, b_rest, Wr1, br1, Wr2, br2):
    row = edge_index[0]
    col = edge_index[1]
    pad = _EPAD - _E
    row_p = jnp.concatenate([row, jnp.zeros((pad,), jnp.int32)])
    col_p = jnp.concatenate([col, jnp.full((pad,), _N, jnp.int32)])
    pk = (row_p | (col_p << 14)).reshape(_NS * _NBLK_T, _BLK)

    deg = _sc_degree(pk).reshape(2, _N, 128)
    u, dis = _tc_prep(x, W0[:_DIN], deg)

    for l in range(_L):
        s = _sc_scatter(u.reshape(4 * _N, 128), pk).reshape(4, _N, 128)
        b_l = b0 if l == 0 else b_rest[l - 1]
        if l < _L - 1:
            u = _tc_layer(s, u, dis, b_l, W_rest[l].reshape(4, 128, _H))
        else:
            out = _tc_final(s, u, dis, b_l, Wr1.reshape(4, 128, _H), br1,
                            Wr2, br2)
    return out
